# Initial kernel scaffold; baseline (speedup 1.0000x reference)
import functools

import jax
import jax.numpy as jnp
from jax import lax
from jax.experimental import pallas as pl
from jax.experimental.pallas import tpu as pltpu
from jax.experimental.pallas import tpu_sc as plsc

N_NODES = 10000
N_EDGES = 320000
F = 128
FH = 64
NC = 2
NS = 16
L = 16
EPT = N_EDGES // NS
K = 80
NCHUNK = EPT // K
RPT = N_NODES // NS
RCH = 125
NRCH = RPT // RCH


def _proj_body(x_ref, wt_ref, wb_ref, wf_ref, ba_ref, bf_ref, pd_ref, qf_ref):
    x = x_ref[...]
    pd_ref[0] = jnp.dot(x, wt_ref[...], preferred_element_type=jnp.float32) + ba_ref[...]
    q = jnp.dot(x, wb_ref[...], preferred_element_type=jnp.float32)
    f = jnp.dot(x, wf_ref[...], preferred_element_type=jnp.float32) + bf_ref[...]
    qf_ref[0, :, :FH] = q
    qf_ref[0, :, FH:] = f


def _project(x, Wa, ba, Wf, bf):
    BN = 1000
    NB = N_NODES // BN
    Wt = Wa[:F]
    Wb = Wa[F:]
    ba2 = ba.reshape(1, F)
    bf2 = bf.reshape(1, F)
    return pl.pallas_call(
        _proj_body,
        grid=(NC, NB),
        in_specs=[
            pl.BlockSpec((BN, F), lambda c, i: (i, 0)),
            pl.BlockSpec((F, FH), lambda c, i: (0, c)),
            pl.BlockSpec((F, FH), lambda c, i: (0, c)),
            pl.BlockSpec((F, FH), lambda c, i: (0, c)),
            pl.BlockSpec((1, FH), lambda c, i: (0, c)),
            pl.BlockSpec((1, FH), lambda c, i: (0, c)),
        ],
        out_specs=[
            pl.BlockSpec((1, BN, FH), lambda c, i: (c, i, 0)),
            pl.BlockSpec((1, BN, F), lambda c, i: (c, i, 0)),
        ],
        out_shape=[
            jax.ShapeDtypeStruct((NC, N_NODES, FH), jnp.float32),
            jax.ShapeDtypeStruct((NC, N_NODES, F), jnp.float32),
        ],
    )(x, Wt, Wb, Wf, ba2, bf2)


def _edge_body(pd_hbm, qf_hbm, src_hbm, dst_hbm, out_hbm,
               idx_s_raw, idx_d_raw, idx_s_off, idx_d_off,
               pd_buf, qf_buf, gbuf, dbuf, obuf, accum, sem1, sem2):
    c = lax.axis_index("c")
    s = lax.axis_index("s")
    node_off = c * N_NODES

    zeros = jnp.zeros((L,), jnp.float32)

    def zero_row(i, _):
        for j in range(F // L):
            dbuf[i, pl.ds(L * j, L)] = zeros
        return 0

    lax.fori_loop(0, RCH, zero_row, 0)
    for u in range(NRCH):
        pltpu.sync_copy(dbuf, accum.at[pl.ds(s * RPT + u * RCH, RCH), :])
    plsc.subcore_barrier()

    def chunk_body(t, _):
        base = s * EPT + t * K
        pltpu.sync_copy(src_hbm.at[pl.ds(base, K)], idx_s_raw)
        pltpu.sync_copy(dst_hbm.at[pl.ds(base, K)], idx_d_raw)
        for j in range(K // L):
            sl = pl.ds(L * j, L)
            idx_s_off[sl] = idx_s_raw[sl] + node_off
            idx_d_off[sl] = idx_d_raw[sl] + node_off
        cp1 = pltpu.async_copy(pd_hbm.at[idx_d_off], pd_buf, sem1)
        cp2 = pltpu.async_copy(qf_hbm.at[idx_s_off], qf_buf, sem2)
        cp1.wait()
        cp2.wait()

        def edge_row(i, _):
            for j in range(FH // L):
                sl = pl.ds(L * j, L)
                sh = pl.ds(FH + L * j, L)
                z = pd_buf[i, sl] + qf_buf[i, sl]
                g = jnp.exp(jnp.maximum(z, 0.01 * z))
                gbuf[i, sl] = g
                gbuf[i, sh] = g * qf_buf[i, sh]
            return 0

        lax.fori_loop(0, K, edge_row, 0)
        pltpu.sync_copy(gbuf, accum.at[idx_d_raw], add=True)
        return 0

    lax.fori_loop(0, NCHUNK, chunk_body, 0)
    plsc.subcore_barrier()

    for u in range(NRCH):
        r0 = s * RPT + u * RCH
        pltpu.sync_copy(accum.at[pl.ds(r0, RCH), :], dbuf)

        def drain_row(i, _):
            for j in range(FH // L):
                d = dbuf[i, pl.ds(L * j, L)]
                n = dbuf[i, pl.ds(FH + L * j, L)]
                r = n / (d + 1e-9)
                obuf[i, pl.ds(L * j, L)] = 1.0 / (1.0 + jnp.exp(-r))
            return 0

        lax.fori_loop(0, RCH, drain_row, 0)
        pltpu.sync_copy(obuf, out_hbm.at[c, pl.ds(r0, RCH), :])


_edge_kernel = functools.partial(
    pl.kernel,
    out_type=jax.ShapeDtypeStruct((NC, N_NODES, FH), jnp.float32),
    mesh=plsc.VectorSubcoreMesh(core_axis_name="c", subcore_axis_name="s"),
    scratch_types=[
        pltpu.VMEM((K,), jnp.int32),
        pltpu.VMEM((K,), jnp.int32),
        pltpu.VMEM((K,), jnp.int32),
        pltpu.VMEM((K,), jnp.int32),
        pltpu.VMEM((K, FH), jnp.float32),
        pltpu.VMEM((K, F), jnp.float32),
        pltpu.VMEM((K, F), jnp.float32),
        pltpu.VMEM((RCH, F), jnp.float32),
        pltpu.VMEM((RCH, FH), jnp.float32),
        pltpu.VMEM_SHARED((N_NODES, F), jnp.float32),
        pltpu.SemaphoreType.DMA,
        pltpu.SemaphoreType.DMA,
    ],
)(_edge_body)


def kernel(x, edge_idx, Wa, ba, Wf, bf):
    edge_idx = edge_idx.astype(jnp.int32)
    pd3, qf3 = _project(x, Wa, ba, Wf, bf)
    pd = pd3.reshape(NC * N_NODES, FH)
    qf = qf3.reshape(NC * N_NODES, F)
    out3 = _edge_kernel(pd, qf, edge_idx[0], edge_idx[1])
    return out3.transpose(1, 0, 2).reshape(N_NODES, F)


# speedup vs baseline: 3.1959x; 3.1959x over previous
import functools

import jax
import jax.numpy as jnp
from jax import lax
from jax.experimental import pallas as pl
from jax.experimental.pallas import tpu as pltpu
from jax.experimental.pallas import tpu_sc as plsc

N_NODES = 10000
N_EDGES = 320000
F = 128
FH = 64
NC = 2
NS = 16
L = 16
EPT = N_EDGES // NS
K = 80
NCHUNK = EPT // K
RPT = N_NODES // NS
RCH = 125
NRCH = RPT // RCH


def _proj_body(x_ref, wt_ref, wb_ref, wf_ref, ba_ref, bf_ref, pd_ref, qf_ref):
    x = x_ref[...]
    p = jnp.dot(x, wt_ref[...], preferred_element_type=jnp.float32) + ba_ref[...]
    q = jnp.dot(x, wb_ref[...], preferred_element_type=jnp.float32)
    f = jnp.dot(x, wf_ref[...], preferred_element_type=jnp.float32) + bf_ref[...]
    pd_ref[0] = p[:, :FH]
    pd_ref[1] = p[:, FH:]
    qf_ref[0, :, :FH] = q[:, :FH]
    qf_ref[0, :, FH:] = f[:, :FH]
    qf_ref[1, :, :FH] = q[:, FH:]
    qf_ref[1, :, FH:] = f[:, FH:]


def _project(x, Wa, ba, Wf, bf):
    BN = 1000
    NB = N_NODES // BN
    Wt = Wa[:F]
    Wb = Wa[F:]
    ba2 = ba.reshape(1, F)
    bf2 = bf.reshape(1, F)
    return pl.pallas_call(
        _proj_body,
        grid=(NB,),
        in_specs=[
            pl.BlockSpec((BN, F), lambda i: (i, 0)),
            pl.BlockSpec((F, F), lambda i: (0, 0)),
            pl.BlockSpec((F, F), lambda i: (0, 0)),
            pl.BlockSpec((F, F), lambda i: (0, 0)),
            pl.BlockSpec((1, F), lambda i: (0, 0)),
            pl.BlockSpec((1, F), lambda i: (0, 0)),
        ],
        out_specs=[
            pl.BlockSpec((NC, BN, FH), lambda i: (0, i, 0)),
            pl.BlockSpec((NC, BN, F), lambda i: (0, i, 0)),
        ],
        out_shape=[
            jax.ShapeDtypeStruct((NC, N_NODES, FH), jnp.float32),
            jax.ShapeDtypeStruct((NC, N_NODES, F), jnp.float32),
        ],
    )(x, Wt, Wb, Wf, ba2, bf2)


def _edge_body(pd_hbm, qf_hbm, src_hbm, dst_hbm, out_hbm,
               idx_s_raw, idx_d_raw, idx_s_off, idx_d_off,
               pd_buf, qf_buf, gbuf, dbuf, obuf, accum, sem1, sem2):
    c = lax.axis_index("c")
    s = lax.axis_index("s")
    node_off = c * N_NODES

    zeros = jnp.zeros((L,), jnp.float32)

    def zero_row(i, _):
        for j in range(F // L):
            dbuf[i, pl.ds(L * j, L)] = zeros
        return 0

    lax.fori_loop(0, RCH, zero_row, 0)
    for u in range(NRCH):
        pltpu.sync_copy(dbuf, accum.at[pl.ds(s * RPT + u * RCH, RCH), :])
    plsc.subcore_barrier()

    def chunk_body(t, _):
        base = s * EPT + t * K
        pltpu.sync_copy(src_hbm.at[pl.ds(base, K)], idx_s_raw)
        pltpu.sync_copy(dst_hbm.at[pl.ds(base, K)], idx_d_raw)
        for j in range(K // L):
            sl = pl.ds(L * j, L)
            idx_s_off[sl] = idx_s_raw[sl] + node_off
            idx_d_off[sl] = idx_d_raw[sl] + node_off
        cp1 = pltpu.async_copy(pd_hbm.at[idx_d_off], pd_buf, sem1)
        cp2 = pltpu.async_copy(qf_hbm.at[idx_s_off], qf_buf, sem2)
        cp1.wait()
        cp2.wait()

        def edge_row(i, _):
            for j in range(FH // L):
                sl = pl.ds(L * j, L)
                sh = pl.ds(FH + L * j, L)
                z = pd_buf[i, sl] + qf_buf[i, sl]
                g = jnp.exp(jnp.maximum(z, 0.01 * z))
                gbuf[i, sl] = g
                gbuf[i, sh] = g * qf_buf[i, sh]
            return 0

        lax.fori_loop(0, K, edge_row, 0)
        pltpu.sync_copy(gbuf, accum.at[idx_d_raw], add=True)
        return 0

    lax.fori_loop(0, NCHUNK, chunk_body, 0)
    plsc.subcore_barrier()

    for u in range(NRCH):
        r0 = s * RPT + u * RCH
        pltpu.sync_copy(accum.at[pl.ds(r0, RCH), :], dbuf)

        def drain_row(i, _):
            for j in range(FH // L):
                d = dbuf[i, pl.ds(L * j, L)]
                n = dbuf[i, pl.ds(FH + L * j, L)]
                r = n / (d + 1e-9)
                obuf[pl.ds(i * FH + L * j, L)] = 1.0 / (1.0 + jnp.exp(-r))
            return 0

        lax.fori_loop(0, RCH, drain_row, 0)
        pltpu.sync_copy(obuf, out_hbm.at[pl.ds((c * N_NODES + r0) * FH, RCH * FH)])


_edge_kernel = functools.partial(
    pl.kernel,
    out_type=jax.ShapeDtypeStruct((NC * N_NODES * FH,), jnp.float32),
    mesh=plsc.VectorSubcoreMesh(core_axis_name="c", subcore_axis_name="s"),
    compiler_params=pltpu.CompilerParams(use_tc_tiling_on_sc=False),
    scratch_types=[
        pltpu.VMEM((K,), jnp.int32),
        pltpu.VMEM((K,), jnp.int32),
        pltpu.VMEM((K,), jnp.int32),
        pltpu.VMEM((K,), jnp.int32),
        pltpu.VMEM((K, FH), jnp.float32),
        pltpu.VMEM((K, F), jnp.float32),
        pltpu.VMEM((K, F), jnp.float32),
        pltpu.VMEM((RCH, F), jnp.float32),
        pltpu.VMEM((RCH * FH,), jnp.float32),
        pltpu.VMEM_SHARED((N_NODES, F), jnp.float32),
        pltpu.SemaphoreType.DMA,
        pltpu.SemaphoreType.DMA,
    ],
)(_edge_body)


def kernel(x, edge_idx, Wa, ba, Wf, bf):
    edge_idx = edge_idx.astype(jnp.int32)
    pd3, qf3 = _project(x, Wa, ba, Wf, bf)
    pd = pd3.reshape(NC * N_NODES, FH)
    qf = qf3.reshape(NC * N_NODES, F)
    out3 = _edge_kernel(pd, qf, edge_idx[0], edge_idx[1])
    return out3.reshape(NC, N_NODES, FH).transpose(1, 0, 2).reshape(N_NODES, F)


# speedup vs baseline: 4.0153x; 1.2564x over previous
import functools

import jax
import jax.numpy as jnp
from jax import lax
from jax.experimental import pallas as pl
from jax.experimental.pallas import tpu as pltpu
from jax.experimental.pallas import tpu_sc as plsc

N_NODES = 10000
N_EDGES = 320000
F = 128
FH = 64
NC = 2
NS = 16
L = 16
EPT = N_EDGES // NS
K = 80
NCHUNK = EPT // K
RPT = N_NODES // NS
RCH = 25
NRCH = RPT // RCH


def _proj_body(x_ref, wt_ref, wb_ref, wf_ref, ba_ref, bf_ref, pd_ref, qf_ref):
    x = x_ref[...]
    p = jnp.dot(x, wt_ref[...], preferred_element_type=jnp.float32) + ba_ref[...]
    q = jnp.dot(x, wb_ref[...], preferred_element_type=jnp.float32)
    f = jnp.dot(x, wf_ref[...], preferred_element_type=jnp.float32) + bf_ref[...]
    pd_ref[0] = p[:, :FH]
    pd_ref[1] = p[:, FH:]
    qf_ref[0, :, :FH] = q[:, :FH]
    qf_ref[0, :, FH:] = f[:, :FH]
    qf_ref[1, :, :FH] = q[:, FH:]
    qf_ref[1, :, FH:] = f[:, FH:]


def _project(x, Wa, ba, Wf, bf):
    BN = 1000
    NB = N_NODES // BN
    Wt = Wa[:F]
    Wb = Wa[F:]
    ba2 = ba.reshape(1, F)
    bf2 = bf.reshape(1, F)
    return pl.pallas_call(
        _proj_body,
        grid=(NB,),
        in_specs=[
            pl.BlockSpec((BN, F), lambda i: (i, 0)),
            pl.BlockSpec((F, F), lambda i: (0, 0)),
            pl.BlockSpec((F, F), lambda i: (0, 0)),
            pl.BlockSpec((F, F), lambda i: (0, 0)),
            pl.BlockSpec((1, F), lambda i: (0, 0)),
            pl.BlockSpec((1, F), lambda i: (0, 0)),
        ],
        out_specs=[
            pl.BlockSpec((NC, BN, FH), lambda i: (0, i, 0)),
            pl.BlockSpec((NC, BN, F), lambda i: (0, i, 0)),
        ],
        out_shape=[
            jax.ShapeDtypeStruct((NC, N_NODES, FH), jnp.float32),
            jax.ShapeDtypeStruct((NC, N_NODES, F), jnp.float32),
        ],
    )(x, Wt, Wb, Wf, ba2, bf2)


def _edge_body(pd_hbm, qf_hbm, ei_hbm, out_hbm,
               idx0, idx1, off_s0, off_s1, off_d0, off_d1, raw_d0, raw_d1,
               pd0, pd1, qf0, qf1, gbuf,
               dbuf, obuf, accum,
               sem_p0, sem_p1, sem_q0, sem_q1):
    c = lax.axis_index("c")
    s = lax.axis_index("s")
    node_off = c * N_NODES
    IDX = (idx0, idx1)
    OFF_S = (off_s0, off_s1)
    OFF_D = (off_d0, off_d1)
    RAW_D = (raw_d0, raw_d1)
    PD = (pd0, pd1)
    QF = (qf0, qf1)
    SEM_P = (sem_p0, sem_p1)
    SEM_Q = (sem_q0, sem_q1)

    zeros = jnp.zeros((L,), jnp.float32)

    def zero_row(i, _):
        for j in range(F // L):
            dbuf[i, pl.ds(L * j, L)] = zeros
        return 0

    lax.fori_loop(0, RCH, zero_row, 0)

    def zero_chunk(u, _):
        pltpu.sync_copy(dbuf, accum.at[pl.ds(s * RPT + u * RCH, RCH), :])
        return 0

    lax.fori_loop(0, NRCH, zero_chunk, 0)
    plsc.subcore_barrier()

    def issue(t, b):
        base = s * EPT + t * K
        pltpu.sync_copy(ei_hbm.at[:, pl.ds(base, K)], IDX[b])
        for j in range(K // L):
            dsl = pl.ds(L * j, L)
            vd = IDX[b][1, dsl]
            OFF_S[b][dsl] = IDX[b][0, dsl] + node_off
            RAW_D[b][dsl] = vd
            OFF_D[b][dsl] = vd + node_off
        pltpu.async_copy(pd_hbm.at[OFF_D[b]], PD[b], SEM_P[b])
        pltpu.async_copy(qf_hbm.at[OFF_S[b]], QF[b], SEM_Q[b])

    def process(b):
        pltpu.make_async_copy(pd_hbm.at[OFF_D[b]], PD[b], SEM_P[b]).wait()
        pltpu.make_async_copy(qf_hbm.at[OFF_S[b]], QF[b], SEM_Q[b]).wait()

        def edge_row(i, _):
            for j in range(FH // L):
                sl = pl.ds(L * j, L)
                sh = pl.ds(FH + L * j, L)
                z = PD[b][i, sl] + QF[b][i, sl]
                g = jnp.exp(jnp.maximum(z, 0.01 * z))
                gbuf[i, sl] = g
                gbuf[i, sh] = g * QF[b][i, sh]
            return 0

        lax.fori_loop(0, K, edge_row, 0)
        pltpu.sync_copy(gbuf, accum.at[RAW_D[b]], add=True)

    NPAIR = NCHUNK // 2
    issue(0, 0)

    def pair_body(h, _):
        issue(2 * h + 1, 1)
        process(0)

        @pl.when(h + 1 < NPAIR)
        def _():
            issue(2 * h + 2, 0)

        process(1)
        return 0

    lax.fori_loop(0, NPAIR, pair_body, 0)
    plsc.subcore_barrier()

    for u in range(NRCH):
        r0 = s * RPT + u * RCH
        pltpu.sync_copy(accum.at[pl.ds(r0, RCH), :], dbuf)

        def drain_row(i, _):
            for j in range(FH // L):
                d = dbuf[i, pl.ds(L * j, L)]
                n = dbuf[i, pl.ds(FH + L * j, L)]
                r = n / (d + 1e-9)
                obuf[pl.ds(i * FH + L * j, L)] = 1.0 / (1.0 + jnp.exp(-r))
            return 0

        lax.fori_loop(0, RCH, drain_row, 0)
        pltpu.sync_copy(obuf, out_hbm.at[pl.ds((c * N_NODES + r0) * FH, RCH * FH)])


_edge_kernel = functools.partial(
    pl.kernel,
    out_type=jax.ShapeDtypeStruct((NC * N_NODES * FH,), jnp.float32),
    mesh=plsc.VectorSubcoreMesh(core_axis_name="c", subcore_axis_name="s"),
    compiler_params=pltpu.CompilerParams(use_tc_tiling_on_sc=False),
    scratch_types=[
        pltpu.VMEM((2, K), jnp.int32),
        pltpu.VMEM((2, K), jnp.int32),
        pltpu.VMEM((K,), jnp.int32),
        pltpu.VMEM((K,), jnp.int32),
        pltpu.VMEM((K,), jnp.int32),
        pltpu.VMEM((K,), jnp.int32),
        pltpu.VMEM((K,), jnp.int32),
        pltpu.VMEM((K,), jnp.int32),
        pltpu.VMEM((K, FH), jnp.float32),
        pltpu.VMEM((K, FH), jnp.float32),
        pltpu.VMEM((K, F), jnp.float32),
        pltpu.VMEM((K, F), jnp.float32),
        pltpu.VMEM((K, F), jnp.float32),
        pltpu.VMEM((RCH, F), jnp.float32),
        pltpu.VMEM((RCH * FH,), jnp.float32),
        pltpu.VMEM_SHARED((N_NODES, F), jnp.float32),
        pltpu.SemaphoreType.DMA,
        pltpu.SemaphoreType.DMA,
        pltpu.SemaphoreType.DMA,
        pltpu.SemaphoreType.DMA,
    ],
)(_edge_body)


def kernel(x, edge_idx, Wa, ba, Wf, bf):
    edge_idx = edge_idx.astype(jnp.int32)
    pd3, qf3 = _project(x, Wa, ba, Wf, bf)
    pd = pd3.reshape(NC * N_NODES, FH)
    qf = qf3.reshape(NC * N_NODES, F)
    out3 = _edge_kernel(pd, qf, edge_idx)
    return out3.reshape(NC, N_NODES, FH).transpose(1, 0, 2).reshape(N_NODES, F)
